# baseline (speedup 1.0000x reference)
import functools

import jax
import jax.numpy as jnp
from jax import lax
from jax.experimental import pallas as pl
from jax.experimental.pallas import tpu as pltpu
from jax.experimental.pallas import tpu_sc as plsc

B, S = 1024, 50
MWL, CFD, EMB = 20, 4, 32
IDX_PER_PAIR = MWL * CFD
EXTRA = 16
FEAT = IDX_PER_PAIR + EXTRA
OUT_W = MWL * EMB + EXTRA

NC, NS = 2, 16
NW = NC * NS
B_PER_W = B // NW

C = 10
CHUNKS = B_PER_W * (S // C)
ROWS = C * IDX_PER_PAIR
GCH = ROWS // 128
GTAIL = ROWS - GCH * 128


def kernel(batch_features, embedding_weight):
    mesh = plsc.VectorSubcoreMesh(core_axis_name="c", subcore_axis_name="s")

    @functools.partial(
        pl.kernel,
        mesh=mesh,
        out_type=jax.ShapeDtypeStruct((B, S, OUT_W), jnp.float32),
        compiler_params=pltpu.CompilerParams(use_tc_tiling_on_sc=False),
        scratch_types=[
            pltpu.VMEM((C, FEAT), jnp.float32),
            pltpu.VMEM((C, FEAT), jnp.float32),
            pltpu.VMEM((GCH + 1, 128), jnp.int32),
            pltpu.VMEM((GCH + 1, 128), jnp.int32),
            pltpu.VMEM((ROWS, EMB), jnp.float32),
            pltpu.VMEM((ROWS, EMB), jnp.float32),
            pltpu.VMEM((C, OUT_W), jnp.float32),
            pltpu.VMEM((C, OUT_W), jnp.float32),
            pltpu.SemaphoreType.DMA,
            pltpu.SemaphoreType.DMA,
            pltpu.SemaphoreType.DMA,
            pltpu.SemaphoreType.DMA,
        ],
    )
    def k(feats_hbm, table_hbm, out_hbm,
          f0, f1, i0, i1, r0, r1, o0, o1, sg0, sg1, so0, so1):
        fv, iv, rv, ov = (f0, f1), (i0, i1), (r0, r1), (o0, o1)
        sg, so = (sg0, sg1), (so0, so1)

        wid = lax.axis_index("s") * NC + lax.axis_index("c")
        wb = wid * B_PER_W

        def streams(p):
            ops = []
            for j in range(GCH):
                ops.append((iv[p].at[j], rv[p].at[pl.ds(j * 128, 128)]))
            ops.append((iv[p].at[GCH, pl.ds(0, GTAIL)],
                        rv[p].at[pl.ds(GCH * 128, GTAIL)]))
            return ops

        def fire(p, b, s):
            pltpu.sync_copy(feats_hbm.at[b, pl.ds(s, C)], fv[p])
            for pp in range(C):
                for kk in range(IDX_PER_PAIR // 16):
                    l = pp * IDX_PER_PAIR + kk * 16
                    v = fv[p][pp, pl.ds(kk * 16, 16)].astype(jnp.int32)
                    iv[p][l // 128, pl.ds(l % 128, 16)] = v
            for src, dst in streams(p):
                pltpu.async_copy(table_hbm.at[src], dst, sg[p])

        def drain(p):
            for src, dst in streams(p):
                pltpu.make_async_copy(table_hbm.at[src], dst, sg[p]).wait()

        def pool_and_out(p, b, s, do_wait):
            @pl.when(do_wait)
            def _():
                pltpu.make_async_copy(
                    ov[p], out_hbm.at[b, pl.ds(s, C)], so[p]).wait()

            def poolbody(pp, c2):
                rbase = pp * IDX_PER_PAIR
                for g2 in range(MWL):
                    for hh in range(2):
                        cs = pl.ds(hh * 16, 16)
                        r = (
                            rv[p][rbase + g2 * 4 + 0, cs]
                            + rv[p][rbase + g2 * 4 + 1, cs]
                            + rv[p][rbase + g2 * 4 + 2, cs]
                            + rv[p][rbase + g2 * 4 + 3, cs]
                        )
                        ov[p][pp, pl.ds(g2 * EMB + hh * 16, 16)] = r
                ov[p][pp, pl.ds(MWL * EMB, 16)] = fv[p][pp, pl.ds(IDX_PER_PAIR, 16)]
                return c2

            lax.fori_loop(0, C, poolbody, 0)
            pltpu.async_copy(ov[p], out_hbm.at[b, pl.ds(s, C)], so[p])

        def adv(b, s):
            s2 = s + C
            w = (s2 >= S).astype(jnp.int32)
            return b + w, s2 - S * w

        fire(0, wb, 0)

        def body(u, carry):
            b0, s0 = carry
            b1, s1 = adv(b0, s0)
            b2, s2 = adv(b1, s1)
            b2 = lax.min(b2, B - 1)

            fire(1, b1, s1)
            drain(0)
            pool_and_out(0, b0, s0, u > 0)
            fire(0, b2, s2)
            drain(1)
            pool_and_out(1, b1, s1, u > 0)
            return (b2, s2)

        lax.fori_loop(0, CHUNKS // 2, body,
                      (jnp.int32(wb), jnp.int32(0)))

        drain(0)
        pltpu.make_async_copy(o0, out_hbm.at[wb, pl.ds(0, C)], so0).wait()
        pltpu.make_async_copy(o1, out_hbm.at[wb, pl.ds(0, C)], so1).wait()

    return k(batch_features, embedding_weight)


# speedup vs baseline: 1.0061x; 1.0061x over previous
import functools

import jax
import jax.numpy as jnp
from jax import lax
from jax.experimental import pallas as pl
from jax.experimental.pallas import tpu as pltpu
from jax.experimental.pallas import tpu_sc as plsc

B, S = 1024, 50
MWL, CFD, EMB = 20, 4, 32
IDX_PER_PAIR = MWL * CFD
EXTRA = 16
FEAT = IDX_PER_PAIR + EXTRA
OUT_W = MWL * EMB + EXTRA

NC, NS = 2, 16
NW = NC * NS

P = B * S
C = 16
ROWS = C * IDX_PER_PAIR
GCH = ROWS // 128
TOTAL_CHUNKS = P // C
W_CHUNKS = TOTAL_CHUNKS // NW


def kernel(batch_features, embedding_weight):
    idx3 = (
        batch_features[:, :, :IDX_PER_PAIR]
        .astype(jnp.int32)
        .reshape(TOTAL_CHUNKS, GCH, 128)
    )
    extras = batch_features[:, :, IDX_PER_PAIR:].reshape(P, EXTRA)

    mesh = plsc.VectorSubcoreMesh(core_axis_name="c", subcore_axis_name="s")

    @functools.partial(
        pl.kernel,
        mesh=mesh,
        out_type=jax.ShapeDtypeStruct((P, OUT_W), jnp.float32),
        compiler_params=pltpu.CompilerParams(use_tc_tiling_on_sc=False),
        scratch_types=[
            pltpu.VMEM((GCH, 128), jnp.int32),
            pltpu.VMEM((GCH, 128), jnp.int32),
            pltpu.VMEM((ROWS, EMB), jnp.float32),
            pltpu.VMEM((ROWS, EMB), jnp.float32),
            pltpu.VMEM((C, OUT_W), jnp.float32),
            pltpu.VMEM((C, OUT_W), jnp.float32),
            pltpu.SemaphoreType.DMA,
            pltpu.SemaphoreType.DMA,
            pltpu.SemaphoreType.DMA,
            pltpu.SemaphoreType.DMA,
            pltpu.SemaphoreType.DMA,
            pltpu.SemaphoreType.DMA,
        ],
    )
    def k(idx_hbm, extras_hbm, table_hbm, out_hbm,
          i0, i1, r0, r1, o0, o1, sg0, sg1, se0, se1, so0, so1):
        iv, rv, ov = (i0, i1), (r0, r1), (o0, o1)
        sg, se, so = (sg0, sg1), (se0, se1), (so0, so1)

        wid = lax.axis_index("s") * NC + lax.axis_index("c")
        wcbase = wid * W_CHUNKS

        def fire(p, wc):
            pltpu.sync_copy(idx_hbm.at[wc], iv[p])
            for j in range(GCH):
                pltpu.async_copy(
                    table_hbm.at[iv[p].at[j]],
                    rv[p].at[pl.ds(j * 128, 128)],
                    sg[p],
                )

        def drain(p):
            for j in range(GCH):
                pltpu.make_async_copy(
                    table_hbm.at[iv[p].at[j]],
                    rv[p].at[pl.ds(j * 128, 128)],
                    sg[p],
                ).wait()

        def pool_and_out(p, wc, do_wait):
            pair = wc * C

            @pl.when(do_wait)
            def _():
                pltpu.make_async_copy(
                    ov[p], out_hbm.at[pl.ds(pair, C)], so[p]).wait()

            ecp = pltpu.make_async_copy(
                extras_hbm.at[pl.ds(pair, C)],
                ov[p].at[:, pl.ds(MWL * EMB, EXTRA)],
                se[p],
            )
            ecp.start()

            def poolbody(pp, c2):
                rbase = pp * IDX_PER_PAIR
                for g2 in range(MWL):
                    for hh in range(2):
                        cs = pl.ds(hh * 16, 16)
                        r = (
                            rv[p][rbase + g2 * 4 + 0, cs]
                            + rv[p][rbase + g2 * 4 + 1, cs]
                            + rv[p][rbase + g2 * 4 + 2, cs]
                            + rv[p][rbase + g2 * 4 + 3, cs]
                        )
                        ov[p][pp, pl.ds(g2 * EMB + hh * 16, 16)] = r
                return c2

            lax.fori_loop(0, C, poolbody, 0)
            ecp.wait()
            pltpu.async_copy(ov[p], out_hbm.at[pl.ds(pair, C)], so[p])

        fire(0, wcbase)

        def body(u, carry):
            wc0 = wcbase + 2 * u
            wc1 = wc0 + 1
            wc2 = lax.min(wc0 + 2, TOTAL_CHUNKS - 1)

            fire(1, wc1)
            drain(0)
            pool_and_out(0, wc0, u > 0)
            fire(0, wc2)
            drain(1)
            pool_and_out(1, wc1, u > 0)
            return carry

        lax.fori_loop(0, W_CHUNKS // 2, body, 0)

        drain(0)
        pltpu.make_async_copy(o0, out_hbm.at[pl.ds(0, C)], so0).wait()
        pltpu.make_async_copy(o1, out_hbm.at[pl.ds(0, C)], so1).wait()

    out = k(idx3, extras, embedding_weight)
    return out.reshape(B, S, OUT_W)
